# Initial kernel scaffold; baseline (speedup 1.0000x reference)
import jax
import jax.numpy as jnp
from jax.experimental import pallas as pl


def kernel(x, edge_index, hyperedge_index, W1, b1, W2, b2, W_in, b_in, W_e, b_e, W_v, b_v, W_out, b_out, W_lp, b_lp):
    raise NotImplementedError("write your pallas kernel here")


# speedup vs baseline: 7.9758x; 7.9758x over previous
import functools

import jax
import jax.numpy as jnp
from jax import lax
from jax.experimental import pallas as pl
from jax.experimental.pallas import tpu as pltpu
from jax.experimental.pallas import tpu_sc as plsc

N = 10000
NP = 10240
NH = 5000
NHP = 5120
E = 320000
DIM = 64
NCORES = 2
NSUB = 16
NW = NCORES * NSUB
EPW = E // NW
C = 80
NCHUNK = EPW // C

_f32 = jnp.float32


def _mesh():
    return plsc.VectorSubcoreMesh(core_axis_name="c", subcore_axis_name="s")


def _sc_counts(dst, hhe, hnode, zeros640, onesC):
    out_type = [
        jax.ShapeDtypeStruct((NCORES * NP,), _f32),
        jax.ShapeDtypeStruct((NCORES * NHP,), _f32),
        jax.ShapeDtypeStruct((NCORES * NP,), _f32),
    ]

    @functools.partial(
        pl.kernel,
        out_type=out_type,
        mesh=_mesh(),
        scratch_types=[
            pltpu.VMEM((C,), jnp.int32),
            pltpu.VMEM((C,), _f32),
            pltpu.VMEM((640,), _f32),
            pltpu.VMEM_SHARED((NP,), _f32),
            pltpu.VMEM_SHARED((NHP,), _f32),
            pltpu.VMEM_SHARED((NP,), _f32),
        ],
    )
    def k(dst_h, hhe_h, hnode_h, z_h, o_h, deg_o, che_o, cnode_o,
          idx_v, ones_v, stage_v, acc_deg, acc_he, acc_node):
        c = lax.axis_index("c")
        s = lax.axis_index("s")
        wid = s * NCORES + c
        pltpu.sync_copy(z_h, stage_v)
        pltpu.sync_copy(o_h, ones_v)
        pltpu.sync_copy(stage_v, acc_deg.at[pl.ds(s * 640, 640)])
        pltpu.sync_copy(stage_v.at[pl.ds(0, 320)], acc_he.at[pl.ds(s * 320, 320)])
        pltpu.sync_copy(stage_v, acc_node.at[pl.ds(s * 640, 640)])
        plsc.subcore_barrier()
        base = wid * EPW

        def body(kk, _):
            off = pl.multiple_of(base + kk * C, 8)
            pltpu.sync_copy(dst_h.at[pl.ds(off, C)], idx_v)
            pltpu.sync_copy(ones_v, acc_deg.at[idx_v], add=True)
            pltpu.sync_copy(hhe_h.at[pl.ds(off, C)], idx_v)
            pltpu.sync_copy(ones_v, acc_he.at[idx_v], add=True)
            pltpu.sync_copy(hnode_h.at[pl.ds(off, C)], idx_v)
            pltpu.sync_copy(ones_v, acc_node.at[idx_v], add=True)
            return 0

        lax.fori_loop(0, NCHUNK, body, 0)
        plsc.subcore_barrier()
        pltpu.sync_copy(acc_deg.at[pl.ds(s * 640, 640)], stage_v)
        pltpu.sync_copy(stage_v, deg_o.at[pl.ds(pl.multiple_of(c * NP + s * 640, 8), 640)])
        pltpu.sync_copy(acc_he.at[pl.ds(s * 320, 320)], stage_v.at[pl.ds(0, 320)])
        pltpu.sync_copy(stage_v.at[pl.ds(0, 320)],
                        che_o.at[pl.ds(pl.multiple_of(c * NHP + s * 320, 8), 320)])
        pltpu.sync_copy(acc_node.at[pl.ds(s * 640, 640)], stage_v)
        pltpu.sync_copy(stage_v, cnode_o.at[pl.ds(pl.multiple_of(c * NP + s * 640, 8), 640)])

    return k(dst, hhe, hnode, zeros640, onesC)


def _sc_scatter2(t1, g1, d1, nacc1, t2, g2, d2, nacc2, zeros2d):
    out_type = [
        jax.ShapeDtypeStruct((NCORES, nacc1, DIM), _f32),
        jax.ShapeDtypeStruct((NCORES, nacc2, DIM), _f32),
    ]
    rpt1 = nacc1 // NSUB
    rpt2 = nacc2 // NSUB

    @functools.partial(
        pl.kernel,
        out_type=out_type,
        mesh=_mesh(),
        compiler_params=pltpu.CompilerParams(use_tc_tiling_on_sc=False),
        scratch_types=[
            pltpu.VMEM((C,), jnp.int32),
            pltpu.VMEM((C,), jnp.int32),
            pltpu.VMEM((C, DIM), _f32),
            pltpu.VMEM((C, DIM), _f32),
            pltpu.VMEM_SHARED((nacc1, DIM), _f32),
            pltpu.VMEM_SHARED((nacc2, DIM), _f32),
            pltpu.SemaphoreType.DMA,
        ],
    )
    def k(t1_h, g1_h, d1_h, t2_h, g2_h, d2_h, z_h, o1, o2,
          gidx, didx, rows, zb, acc1, acc2, sem):
        c = lax.axis_index("c")
        s = lax.axis_index("s")
        wid = s * NCORES + c
        pltpu.sync_copy(z_h, zb)
        for j in range(rpt1 // C):
            pltpu.sync_copy(zb, acc1.at[pl.ds(s * rpt1 + j * C, C)])
        for j in range(rpt2 // C):
            pltpu.sync_copy(zb, acc2.at[pl.ds(s * rpt2 + j * C, C)])
        plsc.subcore_barrier()
        base = wid * EPW
        for t_h, g_h, d_h, acc in ((t1_h, g1_h, d1_h, acc1),
                                   (t2_h, g2_h, d2_h, acc2)):
            def body(kk, _, t_h=t_h, g_h=g_h, d_h=d_h, acc=acc):
                off = pl.multiple_of(base + kk * C, 8)
                pltpu.sync_copy(g_h.at[pl.ds(off, C)], gidx)
                pltpu.sync_copy(d_h.at[pl.ds(off, C)], didx)
                pltpu.async_copy(t_h.at[gidx], rows, sem).wait()
                pltpu.sync_copy(rows, acc.at[didx], add=True)
                return 0

            lax.fori_loop(0, NCHUNK, body, 0)
        plsc.subcore_barrier()
        for acc, o_h, rpt in ((acc1, o1, rpt1), (acc2, o2, rpt2)):
            for j in range(rpt // C):
                b0 = pl.multiple_of(s * rpt + j * C, 8)
                pltpu.sync_copy(acc.at[pl.ds(b0, C)], zb)
                pltpu.sync_copy(zb, o_h.at[c, pl.ds(b0, C)])

    return k(t1, g1, d1, t2, g2, d2, zeros2d)


def _tc_dense1(x, degT, W1, W_in, b_in2):
    BR = 2000

    def body(x_r, deg_r, W1_r, Win_r, bin_r, y1_r, xw1_r, hin_r):
        xb = x_r[...]
        xw1 = jnp.dot(xb, W1_r[...], preferred_element_type=_f32)
        deg = jnp.sum(deg_r[...], axis=1, keepdims=True) + 1.0
        dinv = lax.rsqrt(deg)
        y1_r[...] = xw1 * dinv
        xw1_r[...] = xw1
        hin_r[...] = jnp.maximum(
            jnp.dot(xb, Win_r[...], preferred_element_type=_f32) + bin_r[...], 0.0)

    return pl.pallas_call(
        body,
        grid=(N // BR,),
        in_specs=[
            pl.BlockSpec((BR, 128), lambda i: (i, 0)),
            pl.BlockSpec((BR, NCORES), lambda i: (i, 0)),
            pl.BlockSpec((128, DIM), lambda i: (0, 0)),
            pl.BlockSpec((128, DIM), lambda i: (0, 0)),
            pl.BlockSpec((1, DIM), lambda i: (0, 0)),
        ],
        out_specs=[pl.BlockSpec((BR, DIM), lambda i: (i, 0))] * 3,
        out_shape=[jax.ShapeDtypeStruct((N, DIM), _f32)] * 3,
    )(x, degT, W1, W_in, b_in2)


def _tc_dense2(S1, xw1, degT, b1_2):
    BR = 2000

    def body(S1_r, xw1_r, deg_r, b1_r, h_r, y2_r):
        Ss = S1_r[0] + S1_r[1]
        deg = jnp.sum(deg_r[...], axis=1, keepdims=True) + 1.0
        dinv = lax.rsqrt(deg)
        h = jnp.maximum(dinv * Ss + dinv * dinv * xw1_r[...] + b1_r[...], 0.0)
        h_r[...] = h
        y2_r[...] = h * dinv

    return pl.pallas_call(
        body,
        grid=(N // BR,),
        in_specs=[
            pl.BlockSpec((NCORES, BR, DIM), lambda i: (0, i, 0)),
            pl.BlockSpec((BR, DIM), lambda i: (i, 0)),
            pl.BlockSpec((BR, NCORES), lambda i: (i, 0)),
            pl.BlockSpec((1, DIM), lambda i: (0, 0)),
        ],
        out_specs=[pl.BlockSpec((BR, DIM), lambda i: (i, 0))] * 2,
        out_shape=[jax.ShapeDtypeStruct((N, DIM), _f32)] * 2,
    )(S1, xw1, degT, b1_2)


def _tc_dense3(She, cheT, W_e, b_e2):
    def body(S_r, c_r, We_r, be_r, e2_r):
        Ss = S_r[0] + S_r[1]
        cnt = jnp.sum(c_r[...], axis=1, keepdims=True)
        e = Ss / jnp.maximum(cnt, 1.0)
        e2_r[...] = jnp.maximum(
            jnp.dot(e, We_r[...], preferred_element_type=_f32) + be_r[...], 0.0)

    return pl.pallas_call(
        body,
        grid=(1,),
        in_specs=[
            pl.BlockSpec((NCORES, NHP, DIM), lambda i: (0, 0, 0)),
            pl.BlockSpec((NHP, NCORES), lambda i: (0, 0)),
            pl.BlockSpec((DIM, DIM), lambda i: (0, 0)),
            pl.BlockSpec((1, DIM), lambda i: (0, 0)),
        ],
        out_specs=pl.BlockSpec((NHP, DIM), lambda i: (0, 0)),
        out_shape=jax.ShapeDtypeStruct((NHP, DIM), _f32),
    )(She, cheT, W_e, b_e2)


def _tc_final(S2, h, degT, Sm, cnodeT, h_in,
              W2, b2_2, W_v, b_v2, W_out, b_out2, W_lp, b_lp2):
    BR = 2000

    def body(S2_r, h_r, deg_r, Sm_r, cn_r, hin_r,
             W2_r, b2_r, Wv_r, bv_r, Wo_r, bo_r, Wlp_r, blp_r, out_r):
        deg = jnp.sum(deg_r[...], axis=1, keepdims=True) + 1.0
        dinv = lax.rsqrt(deg)
        agg2 = dinv * (S2_r[0] + S2_r[1]) + dinv * dinv * h_r[...]
        x_gnn = jnp.dot(agg2, W2_r[...], preferred_element_type=_f32) + b2_r[...]
        cnt = jnp.sum(cn_r[...], axis=1, keepdims=True)
        m = (Sm_r[0] + Sm_r[1]) / jnp.maximum(cnt, 1.0)
        h2 = jnp.maximum(
            hin_r[...] + jnp.dot(m, Wv_r[...], preferred_element_type=_f32)
            + bv_r[...], 0.0)
        x_hyper = jnp.dot(h2, Wo_r[...], preferred_element_type=_f32) + bo_r[...]
        Wlp = Wlp_r[...]
        out_r[...] = (jnp.dot(x_gnn, Wlp[0:40], preferred_element_type=_f32)
                      + jnp.dot(x_hyper, Wlp[40:80], preferred_element_type=_f32)
                      + blp_r[...])

    return pl.pallas_call(
        body,
        grid=(N // BR,),
        in_specs=[
            pl.BlockSpec((NCORES, BR, DIM), lambda i: (0, i, 0)),
            pl.BlockSpec((BR, DIM), lambda i: (i, 0)),
            pl.BlockSpec((BR, NCORES), lambda i: (i, 0)),
            pl.BlockSpec((NCORES, BR, DIM), lambda i: (0, i, 0)),
            pl.BlockSpec((BR, NCORES), lambda i: (i, 0)),
            pl.BlockSpec((BR, DIM), lambda i: (i, 0)),
            pl.BlockSpec((DIM, 40), lambda i: (0, 0)),
            pl.BlockSpec((1, 40), lambda i: (0, 0)),
            pl.BlockSpec((DIM, DIM), lambda i: (0, 0)),
            pl.BlockSpec((1, DIM), lambda i: (0, 0)),
            pl.BlockSpec((DIM, 40), lambda i: (0, 0)),
            pl.BlockSpec((1, 40), lambda i: (0, 0)),
            pl.BlockSpec((80, 40), lambda i: (0, 0)),
            pl.BlockSpec((1, 40), lambda i: (0, 0)),
        ],
        out_specs=pl.BlockSpec((BR, 40), lambda i: (i, 0)),
        out_shape=jax.ShapeDtypeStruct((N, 40), _f32),
    )(S2, h, degT, Sm, cnodeT, h_in,
      W2, b2_2, W_v, b_v2, W_out, b_out2, W_lp, b_lp2)


def kernel(x, edge_index, hyperedge_index, W1, b1, W2, b2, W_in, b_in,
           W_e, b_e, W_v, b_v, W_out, b_out, W_lp, b_lp):
    src = edge_index[0]
    dst = edge_index[1]
    hnode = hyperedge_index[0]
    hhe = hyperedge_index[1]

    zeros640 = jnp.zeros((640,), _f32)
    onesC = jnp.ones((C,), _f32)
    zeros2d = jnp.zeros((C, DIM), _f32)

    deg_p, che_p, cnode_p = _sc_counts(dst, hhe, hnode, zeros640, onesC)
    degT = deg_p.reshape(NCORES, NP).T[:N]
    cheT = che_p.reshape(NCORES, NHP).T
    cnodeT = cnode_p.reshape(NCORES, NP).T[:N]

    y1, xw1, h_in = _tc_dense1(x, degT, W1, W_in, b_in.reshape(1, DIM))
    S1, She = _sc_scatter2(y1, src, dst, NP, h_in, hnode, hhe, NHP, zeros2d)
    h, y2 = _tc_dense2(S1[:, :N], xw1, degT, b1.reshape(1, DIM))
    e2 = _tc_dense3(She, cheT, W_e, b_e.reshape(1, DIM))
    S2, Sm = _sc_scatter2(y2, src, dst, NP, e2, hhe, hnode, NP, zeros2d)
    out = _tc_final(S2[:, :N], h, degT, Sm[:, :N], cnodeT, h_in,
                    W2, b2.reshape(1, 40), W_v, b_v.reshape(1, DIM),
                    W_out, b_out.reshape(1, 40), W_lp, b_lp.reshape(1, 40))
    return out


# speedup vs baseline: 27.5458x; 3.4537x over previous
import functools

import jax
import jax.numpy as jnp
from jax import lax
from jax.experimental import pallas as pl
from jax.experimental.pallas import tpu as pltpu
from jax.experimental.pallas import tpu_sc as plsc

N = 10000
NP = 10240
NH = 5000
NHP = 5120
E = 320000
DIM = 64
NCORES = 2
NSUB = 16
NW = NCORES * NSUB
EPW = E // NW
C = 80
NCHUNK = EPW // C

_f32 = jnp.float32


def _mesh():
    return plsc.VectorSubcoreMesh(core_axis_name="c", subcore_axis_name="s")


def _sc_counts(dst, hhe, hnode, zeros640, onesC):
    out_type = [
        jax.ShapeDtypeStruct((NCORES * NP,), _f32),
        jax.ShapeDtypeStruct((NCORES * NHP,), _f32),
        jax.ShapeDtypeStruct((NCORES * NP,), _f32),
    ]

    @functools.partial(
        pl.kernel,
        out_type=out_type,
        mesh=_mesh(),
        scratch_types=[
            pltpu.VMEM((EPW,), jnp.int32),
            pltpu.VMEM((C,), _f32),
            pltpu.VMEM((640,), _f32),
            pltpu.VMEM_SHARED((NP,), _f32),
            pltpu.VMEM_SHARED((NHP,), _f32),
            pltpu.VMEM_SHARED((NP,), _f32),
            pltpu.SemaphoreType.DMA,
        ],
    )
    def k(dst_h, hhe_h, hnode_h, z_h, o_h, deg_o, che_o, cnode_o,
          idx_v, ones_v, stage_v, acc_deg, acc_he, acc_node, csem):
        c = lax.axis_index("c")
        s = lax.axis_index("s")
        wid = s * NCORES + c
        pltpu.sync_copy(z_h, stage_v)
        pltpu.sync_copy(o_h, ones_v)
        pltpu.sync_copy(stage_v, acc_deg.at[pl.ds(s * 640, 640)])
        pltpu.sync_copy(stage_v.at[pl.ds(0, 320)], acc_he.at[pl.ds(s * 320, 320)])
        pltpu.sync_copy(stage_v, acc_node.at[pl.ds(s * 640, 640)])
        plsc.subcore_barrier()
        base = wid * EPW
        NB = 4

        for ih, acc in ((dst_h, acc_deg), (hhe_h, acc_he), (hnode_h, acc_node)):
            pltpu.sync_copy(ih.at[pl.ds(pl.multiple_of(base, 8), EPW)], idx_v)

            def desc(kk, acc=acc):
                off = pl.multiple_of(kk * C, 8)
                return pltpu.make_async_copy(
                    ones_v, acc.at[idx_v.at[pl.ds(off, C)]], csem)

            for b in range(NB):
                desc(b).start(add=True)

            def body(g, _, desc=desc):
                desc(g).wait()
                desc(g + NB).start(add=True)
                return 0

            lax.fori_loop(0, NCHUNK - NB, body, 0)
            for b in range(NB):
                desc(NCHUNK - NB + b).wait()
        plsc.subcore_barrier()
        pltpu.sync_copy(acc_deg.at[pl.ds(s * 640, 640)], stage_v)
        pltpu.sync_copy(stage_v, deg_o.at[pl.ds(pl.multiple_of(c * NP + s * 640, 8), 640)])
        pltpu.sync_copy(acc_he.at[pl.ds(s * 320, 320)], stage_v.at[pl.ds(0, 320)])
        pltpu.sync_copy(stage_v.at[pl.ds(0, 320)],
                        che_o.at[pl.ds(pl.multiple_of(c * NHP + s * 320, 8), 320)])
        pltpu.sync_copy(acc_node.at[pl.ds(s * 640, 640)], stage_v)
        pltpu.sync_copy(stage_v, cnode_o.at[pl.ds(pl.multiple_of(c * NP + s * 640, 8), 640)])

    return k(dst, hhe, hnode, zeros640, onesC)


def _sc_scatter2(t1, g1, d1, nacc1, t2, g2, d2, nacc2, zeros2d):
    out_type = [
        jax.ShapeDtypeStruct((NCORES, nacc1, DIM), _f32),
        jax.ShapeDtypeStruct((NCORES, nacc2, DIM), _f32),
    ]
    rpt1 = nacc1 // NSUB
    rpt2 = nacc2 // NSUB

    @functools.partial(
        pl.kernel,
        out_type=out_type,
        mesh=_mesh(),
        compiler_params=pltpu.CompilerParams(use_tc_tiling_on_sc=False),
        scratch_types=[
            pltpu.VMEM((EPW,), jnp.int32),
            pltpu.VMEM((EPW,), jnp.int32),
            [pltpu.VMEM((C, DIM), _f32)] * 4,
            pltpu.VMEM((C, DIM), _f32),
            pltpu.VMEM_SHARED((nacc1, DIM), _f32),
            pltpu.VMEM_SHARED((nacc2, DIM), _f32),
            [pltpu.SemaphoreType.DMA] * 4,
            [pltpu.SemaphoreType.DMA] * 4,
        ],
    )
    def k(t1_h, g1_h, d1_h, t2_h, g2_h, d2_h, z_h, o1, o2,
          gidx, didx, rows, zb, acc1, acc2, gsem, ssem):
        c = lax.axis_index("c")
        s = lax.axis_index("s")
        wid = s * NCORES + c
        pltpu.sync_copy(z_h, zb)
        for j in range(rpt1 // C):
            pltpu.sync_copy(zb, acc1.at[pl.ds(s * rpt1 + j * C, C)])
        for j in range(rpt2 // C):
            pltpu.sync_copy(zb, acc2.at[pl.ds(s * rpt2 + j * C, C)])
        plsc.subcore_barrier()
        base = pl.multiple_of(wid * EPW, 8)
        NB = 4
        GFULL = NCHUNK // NB
        for t_h, g_h, d_h, acc in ((t1_h, g1_h, d1_h, acc1),
                                   (t2_h, g2_h, d2_h, acc2)):
            pltpu.sync_copy(g_h.at[pl.ds(base, EPW)], gidx)
            pltpu.sync_copy(d_h.at[pl.ds(base, EPW)], didx)

            def gdesc(kk, b, t_h=t_h):
                off = pl.multiple_of(kk * C, 8)
                return pltpu.make_async_copy(
                    t_h.at[gidx.at[pl.ds(off, C)]], rows[b], gsem[b])

            def sdesc(kk, b, acc=acc):
                off = pl.multiple_of(kk * C, 8)
                return pltpu.make_async_copy(
                    rows[b], acc.at[didx.at[pl.ds(off, C)]], ssem[b])

            for b in range(NB):
                gdesc(b, b).start()

            def body(g, _, gdesc=gdesc, sdesc=sdesc):
                for b in range(NB):
                    kk = g * NB + b
                    gdesc(kk, b).wait()
                    sdesc(kk, b).start(add=True)
                    sdesc(kk, b).wait()
                    gdesc(kk + NB, b).start()
                return 0

            lax.fori_loop(0, GFULL - 1, body, 0)
            for b in range(NB):
                kk = (GFULL - 1) * NB + b
                gdesc(kk, b).wait()
                sdesc(kk, b).start(add=True)
            sdesc((GFULL - 1) * NB, 0).wait()
            for kk in range(GFULL * NB, NCHUNK):
                gdesc(kk, 0).start()
                gdesc(kk, 0).wait()
                sdesc(kk, 0).start(add=True)
                sdesc(kk, 0).wait()
            for b in range(1, NB):
                sdesc((GFULL - 1) * NB + b, b).wait()
        plsc.subcore_barrier()
        for acc, o_h, rpt in ((acc1, o1, rpt1), (acc2, o2, rpt2)):
            for j in range(rpt // C):
                b0 = pl.multiple_of(s * rpt + j * C, 8)
                pltpu.sync_copy(acc.at[pl.ds(b0, C)], zb)
                pltpu.sync_copy(zb, o_h.at[c, pl.ds(b0, C)])

    return k(t1, g1, d1, t2, g2, d2, zeros2d)


def _tc_dense1(x, degT, W1, W_in, b_in2):
    BR = 2000

    def body(x_r, deg_r, W1_r, Win_r, bin_r, y1_r, xw1_r, hin_r):
        xb = x_r[...]
        xw1 = jnp.dot(xb, W1_r[...], preferred_element_type=_f32)
        deg = jnp.sum(deg_r[...], axis=1, keepdims=True) + 1.0
        dinv = lax.rsqrt(deg)
        y1_r[...] = xw1 * dinv
        xw1_r[...] = xw1
        hin_r[...] = jnp.maximum(
            jnp.dot(xb, Win_r[...], preferred_element_type=_f32) + bin_r[...], 0.0)

    return pl.pallas_call(
        body,
        grid=(N // BR,),
        in_specs=[
            pl.BlockSpec((BR, 128), lambda i: (i, 0)),
            pl.BlockSpec((BR, NCORES), lambda i: (i, 0)),
            pl.BlockSpec((128, DIM), lambda i: (0, 0)),
            pl.BlockSpec((128, DIM), lambda i: (0, 0)),
            pl.BlockSpec((1, DIM), lambda i: (0, 0)),
        ],
        out_specs=[pl.BlockSpec((BR, DIM), lambda i: (i, 0))] * 3,
        out_shape=[jax.ShapeDtypeStruct((N, DIM), _f32)] * 3,
    )(x, degT, W1, W_in, b_in2)


def _tc_dense2(S1, xw1, degT, b1_2):
    BR = 2000

    def body(S1_r, xw1_r, deg_r, b1_r, h_r, y2_r):
        Ss = S1_r[0] + S1_r[1]
        deg = jnp.sum(deg_r[...], axis=1, keepdims=True) + 1.0
        dinv = lax.rsqrt(deg)
        h = jnp.maximum(dinv * Ss + dinv * dinv * xw1_r[...] + b1_r[...], 0.0)
        h_r[...] = h
        y2_r[...] = h * dinv

    return pl.pallas_call(
        body,
        grid=(N // BR,),
        in_specs=[
            pl.BlockSpec((NCORES, BR, DIM), lambda i: (0, i, 0)),
            pl.BlockSpec((BR, DIM), lambda i: (i, 0)),
            pl.BlockSpec((BR, NCORES), lambda i: (i, 0)),
            pl.BlockSpec((1, DIM), lambda i: (0, 0)),
        ],
        out_specs=[pl.BlockSpec((BR, DIM), lambda i: (i, 0))] * 2,
        out_shape=[jax.ShapeDtypeStruct((N, DIM), _f32)] * 2,
    )(S1, xw1, degT, b1_2)


def _tc_dense3(She, cheT, W_e, b_e2):
    def body(S_r, c_r, We_r, be_r, e2_r):
        Ss = S_r[0] + S_r[1]
        cnt = jnp.sum(c_r[...], axis=1, keepdims=True)
        e = Ss / jnp.maximum(cnt, 1.0)
        e2_r[...] = jnp.maximum(
            jnp.dot(e, We_r[...], preferred_element_type=_f32) + be_r[...], 0.0)

    return pl.pallas_call(
        body,
        grid=(1,),
        in_specs=[
            pl.BlockSpec((NCORES, NHP, DIM), lambda i: (0, 0, 0)),
            pl.BlockSpec((NHP, NCORES), lambda i: (0, 0)),
            pl.BlockSpec((DIM, DIM), lambda i: (0, 0)),
            pl.BlockSpec((1, DIM), lambda i: (0, 0)),
        ],
        out_specs=pl.BlockSpec((NHP, DIM), lambda i: (0, 0)),
        out_shape=jax.ShapeDtypeStruct((NHP, DIM), _f32),
    )(She, cheT, W_e, b_e2)


def _tc_final(S2, h, degT, Sm, cnodeT, h_in,
              W2, b2_2, W_v, b_v2, W_out, b_out2, W_lp, b_lp2):
    BR = 2000

    def body(S2_r, h_r, deg_r, Sm_r, cn_r, hin_r,
             W2_r, b2_r, Wv_r, bv_r, Wo_r, bo_r, Wlp_r, blp_r, out_r):
        deg = jnp.sum(deg_r[...], axis=1, keepdims=True) + 1.0
        dinv = lax.rsqrt(deg)
        agg2 = dinv * (S2_r[0] + S2_r[1]) + dinv * dinv * h_r[...]
        x_gnn = jnp.dot(agg2, W2_r[...], preferred_element_type=_f32) + b2_r[...]
        cnt = jnp.sum(cn_r[...], axis=1, keepdims=True)
        m = (Sm_r[0] + Sm_r[1]) / jnp.maximum(cnt, 1.0)
        h2 = jnp.maximum(
            hin_r[...] + jnp.dot(m, Wv_r[...], preferred_element_type=_f32)
            + bv_r[...], 0.0)
        x_hyper = jnp.dot(h2, Wo_r[...], preferred_element_type=_f32) + bo_r[...]
        Wlp = Wlp_r[...]
        out_r[...] = (jnp.dot(x_gnn, Wlp[0:40], preferred_element_type=_f32)
                      + jnp.dot(x_hyper, Wlp[40:80], preferred_element_type=_f32)
                      + blp_r[...])

    return pl.pallas_call(
        body,
        grid=(N // BR,),
        in_specs=[
            pl.BlockSpec((NCORES, BR, DIM), lambda i: (0, i, 0)),
            pl.BlockSpec((BR, DIM), lambda i: (i, 0)),
            pl.BlockSpec((BR, NCORES), lambda i: (i, 0)),
            pl.BlockSpec((NCORES, BR, DIM), lambda i: (0, i, 0)),
            pl.BlockSpec((BR, NCORES), lambda i: (i, 0)),
            pl.BlockSpec((BR, DIM), lambda i: (i, 0)),
            pl.BlockSpec((DIM, 40), lambda i: (0, 0)),
            pl.BlockSpec((1, 40), lambda i: (0, 0)),
            pl.BlockSpec((DIM, DIM), lambda i: (0, 0)),
            pl.BlockSpec((1, DIM), lambda i: (0, 0)),
            pl.BlockSpec((DIM, 40), lambda i: (0, 0)),
            pl.BlockSpec((1, 40), lambda i: (0, 0)),
            pl.BlockSpec((80, 40), lambda i: (0, 0)),
            pl.BlockSpec((1, 40), lambda i: (0, 0)),
        ],
        out_specs=pl.BlockSpec((BR, 40), lambda i: (i, 0)),
        out_shape=jax.ShapeDtypeStruct((N, 40), _f32),
    )(S2, h, degT, Sm, cnodeT, h_in,
      W2, b2_2, W_v, b_v2, W_out, b_out2, W_lp, b_lp2)


def kernel(x, edge_index, hyperedge_index, W1, b1, W2, b2, W_in, b_in,
           W_e, b_e, W_v, b_v, W_out, b_out, W_lp, b_lp):
    src = edge_index[0]
    dst = edge_index[1]
    hnode = hyperedge_index[0]
    hhe = hyperedge_index[1]

    zeros640 = jnp.zeros((640,), _f32)
    onesC = jnp.ones((C,), _f32)
    zeros2d = jnp.zeros((C, DIM), _f32)

    deg_p, che_p, cnode_p = _sc_counts(dst, hhe, hnode, zeros640, onesC)
    degT = deg_p.reshape(NCORES, NP).T[:N]
    cheT = che_p.reshape(NCORES, NHP).T
    cnodeT = cnode_p.reshape(NCORES, NP).T[:N]

    y1, xw1, h_in = _tc_dense1(x, degT, W1, W_in, b_in.reshape(1, DIM))
    S1, She = _sc_scatter2(y1, src, dst, NP, h_in, hnode, hhe, NHP, zeros2d)
    h, y2 = _tc_dense2(S1[:, :N], xw1, degT, b1.reshape(1, DIM))
    e2 = _tc_dense3(She, cheT, W_e, b_e.reshape(1, DIM))
    S2, Sm = _sc_scatter2(y2, src, dst, NP, e2, hhe, hnode, NP, zeros2d)
    out = _tc_final(S2[:, :N], h, degT, Sm[:, :N], cnodeT, h_in,
                    W2, b2.reshape(1, 40), W_v, b_v.reshape(1, DIM),
                    W_out, b_out.reshape(1, 40), W_lp, b_lp.reshape(1, 40))
    return out


# speedup vs baseline: 27.7200x; 1.0063x over previous
import functools

import jax
import jax.numpy as jnp
from jax import lax
from jax.experimental import pallas as pl
from jax.experimental.pallas import tpu as pltpu
from jax.experimental.pallas import tpu_sc as plsc

N = 10000
NP = 10240
NH = 5000
NHP = 5120
E = 320000
EP = 327680
DIM = 64
NCORES = 2
NSUB = 16
NW = NCORES * NSUB
EPW = EP // NW
C = 80
NCHUNK = EPW // C

_f32 = jnp.float32


def _mesh():
    return plsc.VectorSubcoreMesh(core_axis_name="c", subcore_axis_name="s")


def _sc_counts(dst, hhe, hnode, zeros640, onesC):
    out_type = [
        jax.ShapeDtypeStruct((NCORES * NP,), _f32),
        jax.ShapeDtypeStruct((NCORES * NHP,), _f32),
        jax.ShapeDtypeStruct((NCORES * NP,), _f32),
    ]

    @functools.partial(
        pl.kernel,
        out_type=out_type,
        mesh=_mesh(),
        scratch_types=[
            pltpu.VMEM((EPW,), jnp.int32),
            pltpu.VMEM((C,), _f32),
            pltpu.VMEM((640,), _f32),
            pltpu.VMEM_SHARED((NP,), _f32),
            pltpu.VMEM_SHARED((NHP,), _f32),
            pltpu.VMEM_SHARED((NP,), _f32),
            pltpu.SemaphoreType.DMA,
        ],
    )
    def k(dst_h, hhe_h, hnode_h, z_h, o_h, deg_o, che_o, cnode_o,
          idx_v, ones_v, stage_v, acc_deg, acc_he, acc_node, csem):
        c = lax.axis_index("c")
        s = lax.axis_index("s")
        wid = s * NCORES + c
        pltpu.sync_copy(z_h, stage_v)
        pltpu.sync_copy(o_h, ones_v)
        pltpu.sync_copy(stage_v, acc_deg.at[pl.ds(s * 640, 640)])
        pltpu.sync_copy(stage_v.at[pl.ds(0, 320)], acc_he.at[pl.ds(s * 320, 320)])
        pltpu.sync_copy(stage_v, acc_node.at[pl.ds(s * 640, 640)])
        plsc.subcore_barrier()
        base = wid * EPW
        NB = 4

        for ih, acc in ((dst_h, acc_deg), (hhe_h, acc_he), (hnode_h, acc_node)):
            pltpu.sync_copy(ih.at[pl.ds(pl.multiple_of(base, 8), EPW)], idx_v)

            def desc(kk, acc=acc):
                off = pl.multiple_of(kk * C, 8)
                return pltpu.make_async_copy(
                    ones_v, acc.at[idx_v.at[pl.ds(off, C)]], csem)

            for b in range(NB):
                desc(b).start(add=True)

            def body(g, _, desc=desc):
                desc(g).wait()
                desc(g + NB).start(add=True)
                return 0

            lax.fori_loop(0, NCHUNK - NB, body, 0)
            for b in range(NB):
                desc(NCHUNK - NB + b).wait()
        plsc.subcore_barrier()
        pltpu.sync_copy(acc_deg.at[pl.ds(s * 640, 640)], stage_v)
        pltpu.sync_copy(stage_v, deg_o.at[pl.ds(pl.multiple_of(c * NP + s * 640, 8), 640)])
        pltpu.sync_copy(acc_he.at[pl.ds(s * 320, 320)], stage_v.at[pl.ds(0, 320)])
        pltpu.sync_copy(stage_v.at[pl.ds(0, 320)],
                        che_o.at[pl.ds(pl.multiple_of(c * NHP + s * 320, 8), 320)])
        pltpu.sync_copy(acc_node.at[pl.ds(s * 640, 640)], stage_v)
        pltpu.sync_copy(stage_v, cnode_o.at[pl.ds(pl.multiple_of(c * NP + s * 640, 8), 640)])

    return k(dst, hhe, hnode, zeros640, onesC)


def _sc_scatter2(t1, g1, d1, nacc1, t2, g2, d2, nacc2, zeros2d):
    out_type = [
        jax.ShapeDtypeStruct((NCORES, nacc1, DIM), _f32),
        jax.ShapeDtypeStruct((NCORES, nacc2, DIM), _f32),
    ]
    rpt1 = nacc1 // NSUB
    rpt2 = nacc2 // NSUB

    @functools.partial(
        pl.kernel,
        out_type=out_type,
        mesh=_mesh(),
        compiler_params=pltpu.CompilerParams(use_tc_tiling_on_sc=False),
        scratch_types=[
            pltpu.VMEM((EPW,), jnp.int32),
            pltpu.VMEM((EPW,), jnp.int32),
            [pltpu.VMEM((C, DIM), _f32)] * 8,
            pltpu.VMEM((C, DIM), _f32),
            pltpu.VMEM_SHARED((nacc1, DIM), _f32),
            pltpu.VMEM_SHARED((nacc2, DIM), _f32),
            [pltpu.SemaphoreType.DMA] * 8,
            [pltpu.SemaphoreType.DMA] * 8,
        ],
    )
    def k(t1_h, g1_h, d1_h, t2_h, g2_h, d2_h, z_h, o1, o2,
          gidx, didx, rows, zb, acc1, acc2, gsem, ssem):
        c = lax.axis_index("c")
        s = lax.axis_index("s")
        wid = s * NCORES + c
        pltpu.sync_copy(z_h, zb)
        for j in range(rpt1 // C):
            pltpu.sync_copy(zb, acc1.at[pl.ds(s * rpt1 + j * C, C)])
        for j in range(rpt2 // C):
            pltpu.sync_copy(zb, acc2.at[pl.ds(s * rpt2 + j * C, C)])
        plsc.subcore_barrier()
        base = pl.multiple_of(wid * EPW, 8)
        NB = 8
        D = 4
        GRP = NCHUNK // NB
        for t_h, g_h, d_h, acc in ((t1_h, g1_h, d1_h, acc1),
                                   (t2_h, g2_h, d2_h, acc2)):
            pltpu.sync_copy(g_h.at[pl.ds(base, EPW)], gidx)
            pltpu.sync_copy(d_h.at[pl.ds(base, EPW)], didx)

            def gdesc(kk, b, t_h=t_h):
                off = pl.multiple_of(kk * C, 8)
                return pltpu.make_async_copy(
                    t_h.at[gidx.at[pl.ds(off, C)]], rows[b], gsem[b])

            def sdesc(kk, b, acc=acc):
                off = pl.multiple_of(kk * C, 8)
                return pltpu.make_async_copy(
                    rows[b], acc.at[didx.at[pl.ds(off, C)]], ssem[b])

            for b in range(NB):
                gdesc(b, b).start()
                if b >= D:
                    gdesc(b - D, b - D).wait()
                    sdesc(b - D, b - D).start(add=True)

            def body(g, _, gdesc=gdesc, sdesc=sdesc):
                for b in range(NB):
                    kk = g * NB + b
                    sdesc(kk - NB, b).wait()
                    gdesc(kk, b).start()
                    gdesc(kk - D, (b - D) % NB).wait()
                    sdesc(kk - D, (b - D) % NB).start(add=True)
                return 0

            lax.fori_loop(1, GRP, body, 0)
            last = (GRP - 1) * NB
            for b in range(D):
                sdesc(last + b, b).wait()
                gdesc(last + NB - D + b, (b - D) % NB).wait()
                sdesc(last + NB - D + b, (b - D) % NB).start(add=True)
            for b in range(D):
                sdesc(last + NB - D + b, (b - D) % NB).wait()
        plsc.subcore_barrier()
        for acc, o_h, rpt in ((acc1, o1, rpt1), (acc2, o2, rpt2)):
            for j in range(rpt // C):
                b0 = pl.multiple_of(s * rpt + j * C, 8)
                pltpu.sync_copy(acc.at[pl.ds(b0, C)], zb)
                pltpu.sync_copy(zb, o_h.at[c, pl.ds(b0, C)])

    return k(t1, g1, d1, t2, g2, d2, zeros2d)


def _tc_dense1(x, degT, W1, W_in, b_in2):
    BR = 2000

    def body(x_r, deg_r, W1_r, Win_r, bin_r, y1_r, xw1_r, hin_r):
        xb = x_r[...]
        xw1 = jnp.dot(xb, W1_r[...], preferred_element_type=_f32)
        deg = jnp.sum(deg_r[...], axis=1, keepdims=True) + 1.0
        dinv = lax.rsqrt(deg)
        y1_r[...] = xw1 * dinv
        xw1_r[...] = xw1
        hin_r[...] = jnp.maximum(
            jnp.dot(xb, Win_r[...], preferred_element_type=_f32) + bin_r[...], 0.0)

    return pl.pallas_call(
        body,
        grid=(N // BR,),
        in_specs=[
            pl.BlockSpec((BR, 128), lambda i: (i, 0)),
            pl.BlockSpec((BR, NCORES), lambda i: (i, 0)),
            pl.BlockSpec((128, DIM), lambda i: (0, 0)),
            pl.BlockSpec((128, DIM), lambda i: (0, 0)),
            pl.BlockSpec((1, DIM), lambda i: (0, 0)),
        ],
        out_specs=[pl.BlockSpec((BR, DIM), lambda i: (i, 0))] * 3,
        out_shape=[jax.ShapeDtypeStruct((N, DIM), _f32)] * 3,
    )(x, degT, W1, W_in, b_in2)


def _tc_dense2(S1, xw1, degT, b1_2):
    BR = 2000

    def body(S1_r, xw1_r, deg_r, b1_r, h_r, y2_r):
        Ss = S1_r[0] + S1_r[1]
        deg = jnp.sum(deg_r[...], axis=1, keepdims=True) + 1.0
        dinv = lax.rsqrt(deg)
        h = jnp.maximum(dinv * Ss + dinv * dinv * xw1_r[...] + b1_r[...], 0.0)
        h_r[...] = h
        y2_r[...] = h * dinv

    return pl.pallas_call(
        body,
        grid=(N // BR,),
        in_specs=[
            pl.BlockSpec((NCORES, BR, DIM), lambda i: (0, i, 0)),
            pl.BlockSpec((BR, DIM), lambda i: (i, 0)),
            pl.BlockSpec((BR, NCORES), lambda i: (i, 0)),
            pl.BlockSpec((1, DIM), lambda i: (0, 0)),
        ],
        out_specs=[pl.BlockSpec((BR, DIM), lambda i: (i, 0))] * 2,
        out_shape=[jax.ShapeDtypeStruct((N, DIM), _f32)] * 2,
    )(S1, xw1, degT, b1_2)


def _tc_dense3(She, cheT, W_e, b_e2):
    def body(S_r, c_r, We_r, be_r, e2_r):
        Ss = S_r[0] + S_r[1]
        cnt = jnp.sum(c_r[...], axis=1, keepdims=True)
        e = Ss / jnp.maximum(cnt, 1.0)
        e2_r[...] = jnp.maximum(
            jnp.dot(e, We_r[...], preferred_element_type=_f32) + be_r[...], 0.0)

    return pl.pallas_call(
        body,
        grid=(1,),
        in_specs=[
            pl.BlockSpec((NCORES, NHP, DIM), lambda i: (0, 0, 0)),
            pl.BlockSpec((NHP, NCORES), lambda i: (0, 0)),
            pl.BlockSpec((DIM, DIM), lambda i: (0, 0)),
            pl.BlockSpec((1, DIM), lambda i: (0, 0)),
        ],
        out_specs=pl.BlockSpec((NHP, DIM), lambda i: (0, 0)),
        out_shape=jax.ShapeDtypeStruct((NHP, DIM), _f32),
    )(She, cheT, W_e, b_e2)


def _tc_final(S2, h, degT, Sm, cnodeT, h_in,
              W2, b2_2, W_v, b_v2, W_out, b_out2, W_lp, b_lp2):
    BR = 2000

    def body(S2_r, h_r, deg_r, Sm_r, cn_r, hin_r,
             W2_r, b2_r, Wv_r, bv_r, Wo_r, bo_r, Wlp_r, blp_r, out_r):
        deg = jnp.sum(deg_r[...], axis=1, keepdims=True) + 1.0
        dinv = lax.rsqrt(deg)
        agg2 = dinv * (S2_r[0] + S2_r[1]) + dinv * dinv * h_r[...]
        x_gnn = jnp.dot(agg2, W2_r[...], preferred_element_type=_f32) + b2_r[...]
        cnt = jnp.sum(cn_r[...], axis=1, keepdims=True)
        m = (Sm_r[0] + Sm_r[1]) * jnp.where(cnt > 0.0,
                                            1.0 / jnp.maximum(cnt, 1.0), 0.0)
        h2 = jnp.maximum(
            hin_r[...] + jnp.dot(m, Wv_r[...], preferred_element_type=_f32)
            + bv_r[...], 0.0)
        x_hyper = jnp.dot(h2, Wo_r[...], preferred_element_type=_f32) + bo_r[...]
        Wlp = Wlp_r[...]
        out_r[...] = (jnp.dot(x_gnn, Wlp[0:40], preferred_element_type=_f32)
                      + jnp.dot(x_hyper, Wlp[40:80], preferred_element_type=_f32)
                      + blp_r[...])

    return pl.pallas_call(
        body,
        grid=(N // BR,),
        in_specs=[
            pl.BlockSpec((NCORES, BR, DIM), lambda i: (0, i, 0)),
            pl.BlockSpec((BR, DIM), lambda i: (i, 0)),
            pl.BlockSpec((BR, NCORES), lambda i: (i, 0)),
            pl.BlockSpec((NCORES, BR, DIM), lambda i: (0, i, 0)),
            pl.BlockSpec((BR, NCORES), lambda i: (i, 0)),
            pl.BlockSpec((BR, DIM), lambda i: (i, 0)),
            pl.BlockSpec((DIM, 40), lambda i: (0, 0)),
            pl.BlockSpec((1, 40), lambda i: (0, 0)),
            pl.BlockSpec((DIM, DIM), lambda i: (0, 0)),
            pl.BlockSpec((1, DIM), lambda i: (0, 0)),
            pl.BlockSpec((DIM, 40), lambda i: (0, 0)),
            pl.BlockSpec((1, 40), lambda i: (0, 0)),
            pl.BlockSpec((80, 40), lambda i: (0, 0)),
            pl.BlockSpec((1, 40), lambda i: (0, 0)),
        ],
        out_specs=pl.BlockSpec((BR, 40), lambda i: (i, 0)),
        out_shape=jax.ShapeDtypeStruct((N, 40), _f32),
    )(S2, h, degT, Sm, cnodeT, h_in,
      W2, b2_2, W_v, b_v2, W_out, b_out2, W_lp, b_lp2)


def kernel(x, edge_index, hyperedge_index, W1, b1, W2, b2, W_in, b_in,
           W_e, b_e, W_v, b_v, W_out, b_out, W_lp, b_lp):
    npad = EP - E
    ar = jnp.arange(npad, dtype=jnp.int32)
    pad_g = ar % 256
    pad_n = N + (ar % (NP - N))
    pad_h = NH + (ar % (NHP - NH))
    src_g = jnp.concatenate([edge_index[0], pad_g])
    dst_n = jnp.concatenate([edge_index[1], pad_n])
    hnode_g = jnp.concatenate([hyperedge_index[0], pad_g])
    hnode_n = jnp.concatenate([hyperedge_index[0], pad_n])
    hnode_h = jnp.concatenate([hyperedge_index[0], pad_h])
    hhe_g = jnp.concatenate([hyperedge_index[1], pad_g])
    hhe_h = jnp.concatenate([hyperedge_index[1], pad_h])

    zeros640 = jnp.zeros((640,), _f32)
    onesC = jnp.ones((C,), _f32)
    zeros2d = jnp.zeros((C, DIM), _f32)

    deg_p, che_p, cnode_p = _sc_counts(dst_n, hhe_h, hnode_n, zeros640, onesC)
    degT = deg_p.reshape(NCORES, NP).T[:N]
    cheT = che_p.reshape(NCORES, NHP).T
    cnodeT = cnode_p.reshape(NCORES, NP).T[:N]

    y1, xw1, h_in = _tc_dense1(x, degT, W1, W_in, b_in.reshape(1, DIM))
    S1, She = _sc_scatter2(y1, src_g, dst_n, NP, h_in, hnode_g, hhe_h, NHP,
                           zeros2d)
    h, y2 = _tc_dense2(S1[:, :N], xw1, degT, b1.reshape(1, DIM))
    e2 = _tc_dense3(She, cheT, W_e, b_e.reshape(1, DIM))
    S2, Sm = _sc_scatter2(y2, src_g, dst_n, NP, e2, hhe_g, hnode_h, NHP,
                          zeros2d)
    Smp = jnp.concatenate([Sm, jnp.zeros((NCORES, N - NHP, DIM), _f32)],
                          axis=1)
    out = _tc_final(S2[:, :N], h, degT, Smp, cnodeT, h_in,
                    W2, b2.reshape(1, 40), W_v, b_v.reshape(1, DIM),
                    W_out, b_out.reshape(1, 40), W_lp, b_lp.reshape(1, 40))
    return out


# speedup vs baseline: 28.0164x; 1.0107x over previous
import functools

import jax
import jax.numpy as jnp
from jax import lax
from jax.experimental import pallas as pl
from jax.experimental.pallas import tpu as pltpu
from jax.experimental.pallas import tpu_sc as plsc

N = 10000
NP = 10240
NH = 5000
NHP = 5120
E = 320000
EP = 327680
DIM = 64
NCORES = 2
NSUB = 16
NW = NCORES * NSUB
EPW = EP // NW
C = 80
NCHUNK = EPW // C

_f32 = jnp.float32


def _mesh():
    return plsc.VectorSubcoreMesh(core_axis_name="c", subcore_axis_name="s")


def _sc_counts(dst, hhe, hnode, zeros640, onesC):
    out_type = [
        jax.ShapeDtypeStruct((NCORES * NP,), _f32),
        jax.ShapeDtypeStruct((NCORES * NHP,), _f32),
        jax.ShapeDtypeStruct((NCORES * NP,), _f32),
    ]

    @functools.partial(
        pl.kernel,
        out_type=out_type,
        mesh=_mesh(),
        scratch_types=[
            pltpu.VMEM((EPW,), jnp.int32),
            pltpu.VMEM((C,), _f32),
            pltpu.VMEM((640,), _f32),
            pltpu.VMEM_SHARED((NP,), _f32),
            pltpu.VMEM_SHARED((NHP,), _f32),
            pltpu.VMEM_SHARED((NP,), _f32),
            pltpu.SemaphoreType.DMA,
        ],
    )
    def k(dst_h, hhe_h, hnode_h, z_h, o_h, deg_o, che_o, cnode_o,
          idx_v, ones_v, stage_v, acc_deg, acc_he, acc_node, csem):
        c = lax.axis_index("c")
        s = lax.axis_index("s")
        wid = s * NCORES + c
        pltpu.sync_copy(z_h, stage_v)
        pltpu.sync_copy(o_h, ones_v)
        pltpu.sync_copy(stage_v, acc_deg.at[pl.ds(s * 640, 640)])
        pltpu.sync_copy(stage_v.at[pl.ds(0, 320)], acc_he.at[pl.ds(s * 320, 320)])
        pltpu.sync_copy(stage_v, acc_node.at[pl.ds(s * 640, 640)])
        plsc.subcore_barrier()
        base = wid * EPW
        NB = 4

        for ih, acc in ((dst_h, acc_deg), (hhe_h, acc_he), (hnode_h, acc_node)):
            pltpu.sync_copy(ih.at[pl.ds(pl.multiple_of(base, 8), EPW)], idx_v)

            def desc(kk, acc=acc):
                off = pl.multiple_of(kk * C, 8)
                return pltpu.make_async_copy(
                    ones_v, acc.at[idx_v.at[pl.ds(off, C)]], csem)

            for b in range(NB):
                desc(b).start(add=True)

            def body(g, _, desc=desc):
                desc(g).wait()
                desc(g + NB).start(add=True)
                return 0

            lax.fori_loop(0, NCHUNK - NB, body, 0)
            for b in range(NB):
                desc(NCHUNK - NB + b).wait()
        plsc.subcore_barrier()
        pltpu.sync_copy(acc_deg.at[pl.ds(s * 640, 640)], stage_v)
        pltpu.sync_copy(stage_v, deg_o.at[pl.ds(pl.multiple_of(c * NP + s * 640, 8), 640)])
        pltpu.sync_copy(acc_he.at[pl.ds(s * 320, 320)], stage_v.at[pl.ds(0, 320)])
        pltpu.sync_copy(stage_v.at[pl.ds(0, 320)],
                        che_o.at[pl.ds(pl.multiple_of(c * NHP + s * 320, 8), 320)])
        pltpu.sync_copy(acc_node.at[pl.ds(s * 640, 640)], stage_v)
        pltpu.sync_copy(stage_v, cnode_o.at[pl.ds(pl.multiple_of(c * NP + s * 640, 8), 640)])

    return k(dst, hhe, hnode, zeros640, onesC)


def _sc_scatter2(t1, g1, d1, nacc1, t2, g2, d2, nacc2, zeros2d):
    out_type = [
        jax.ShapeDtypeStruct((NCORES, nacc1, DIM), _f32),
        jax.ShapeDtypeStruct((NCORES, nacc2, DIM), _f32),
    ]
    rpt1 = nacc1 // NSUB
    rpt2 = nacc2 // NSUB

    @functools.partial(
        pl.kernel,
        out_type=out_type,
        mesh=_mesh(),
        compiler_params=pltpu.CompilerParams(use_tc_tiling_on_sc=False),
        scratch_types=[
            pltpu.VMEM((EPW,), jnp.int32),
            pltpu.VMEM((EPW,), jnp.int32),
            [pltpu.VMEM((C, DIM), _f32)] * 8,
            pltpu.VMEM((C, DIM), _f32),
            pltpu.VMEM_SHARED((nacc1, DIM), _f32),
            pltpu.VMEM_SHARED((nacc2, DIM), _f32),
            [pltpu.SemaphoreType.DMA] * 8,
            [pltpu.SemaphoreType.DMA] * 8,
        ],
    )
    def k(t1_h, g1_h, d1_h, t2_h, g2_h, d2_h, z_h, o1, o2,
          gidx, didx, rows, zb, acc1, acc2, gsem, ssem):
        c = lax.axis_index("c")
        s = lax.axis_index("s")
        wid = s * NCORES + c
        pltpu.sync_copy(z_h, zb)
        for j in range(rpt1 // C):
            pltpu.sync_copy(zb, acc1.at[pl.ds(s * rpt1 + j * C, C)])
        for j in range(rpt2 // C):
            pltpu.sync_copy(zb, acc2.at[pl.ds(s * rpt2 + j * C, C)])
        plsc.subcore_barrier()
        base = pl.multiple_of(wid * EPW, 8)
        NB = 8
        D = 4
        GRP = NCHUNK // NB
        for t_h, g_h, d_h, acc in ((t1_h, g1_h, d1_h, acc1),
                                   (t2_h, g2_h, d2_h, acc2)):
            pltpu.sync_copy(g_h.at[pl.ds(base, EPW)], gidx)
            pltpu.sync_copy(d_h.at[pl.ds(base, EPW)], didx)

            def gdesc(kk, b, t_h=t_h):
                off = pl.multiple_of(kk * C, 8)
                return pltpu.make_async_copy(
                    t_h.at[gidx.at[pl.ds(off, C)]], rows[b], gsem[b])

            def sdesc(kk, b, acc=acc):
                off = pl.multiple_of(kk * C, 8)
                return pltpu.make_async_copy(
                    rows[b], acc.at[didx.at[pl.ds(off, C)]], ssem[b])

            for b in range(NB):
                gdesc(b, b).start()
                if b >= D:
                    gdesc(b - D, b - D).wait()
                    sdesc(b - D, b - D).start(add=True)

            def body(g, _, gdesc=gdesc, sdesc=sdesc):
                for b in range(NB):
                    kk = g * NB + b
                    sdesc(kk - NB, b).wait()
                    gdesc(kk, b).start()
                    gdesc(kk - D, (b - D) % NB).wait()
                    sdesc(kk - D, (b - D) % NB).start(add=True)
                return 0

            lax.fori_loop(1, GRP, body, 0)
            last = (GRP - 1) * NB
            for b in range(D):
                sdesc(last + b, b).wait()
                gdesc(last + NB - D + b, (b - D) % NB).wait()
                sdesc(last + NB - D + b, (b - D) % NB).start(add=True)
            for b in range(D):
                sdesc(last + NB - D + b, (b - D) % NB).wait()
        plsc.subcore_barrier()
        for acc, o_h, rpt in ((acc1, o1, rpt1), (acc2, o2, rpt2)):
            for j in range(rpt // C):
                b0 = pl.multiple_of(s * rpt + j * C, 8)
                pltpu.sync_copy(acc.at[pl.ds(b0, C)], zb)
                pltpu.sync_copy(zb, o_h.at[c, pl.ds(b0, C)])

    return k(t1, g1, d1, t2, g2, d2, zeros2d)


def _tc_dense1(x, degT, W1, W_in, b_in2):
    BR = 2000

    def body(x_r, deg_r, W1_r, Win_r, bin_r, y1_r, xw1_r, hin_r):
        xb = x_r[...]
        xw1 = jnp.dot(xb, W1_r[...], preferred_element_type=_f32)
        deg = jnp.sum(deg_r[...], axis=1, keepdims=True) + 1.0
        dinv = lax.rsqrt(deg)
        y1_r[...] = xw1 * dinv
        xw1_r[...] = xw1
        hin_r[...] = jnp.maximum(
            jnp.dot(xb, Win_r[...], preferred_element_type=_f32) + bin_r[...], 0.0)

    return pl.pallas_call(
        body,
        grid=(N // BR,),
        in_specs=[
            pl.BlockSpec((BR, 128), lambda i: (i, 0)),
            pl.BlockSpec((BR, NCORES), lambda i: (i, 0)),
            pl.BlockSpec((128, DIM), lambda i: (0, 0)),
            pl.BlockSpec((128, DIM), lambda i: (0, 0)),
            pl.BlockSpec((1, DIM), lambda i: (0, 0)),
        ],
        out_specs=[pl.BlockSpec((BR, DIM), lambda i: (i, 0))] * 3,
        out_shape=[jax.ShapeDtypeStruct((N, DIM), _f32)] * 3,
    )(x, degT, W1, W_in, b_in2)


def _tc_dense23(S1, xw1, degT, b1_2, She, cheT, W_e, b_e2):
    BR = 2000
    BH = NHP // (N // BR)

    def body(S1_r, xw1_r, deg_r, b1_r, S_r, c_r, We_r, be_r,
             h_r, y2_r, e2_r):
        Ss = S1_r[0] + S1_r[1]
        deg = jnp.sum(deg_r[...], axis=1, keepdims=True) + 1.0
        dinv = lax.rsqrt(deg)
        h = jnp.maximum(dinv * Ss + dinv * dinv * xw1_r[...] + b1_r[...], 0.0)
        h_r[...] = h
        y2_r[...] = h * dinv
        Se = S_r[0] + S_r[1]
        cnt = jnp.sum(c_r[...], axis=1, keepdims=True)
        e = Se / jnp.maximum(cnt, 1.0)
        e2_r[...] = jnp.maximum(
            jnp.dot(e, We_r[...], preferred_element_type=_f32) + be_r[...], 0.0)

    return pl.pallas_call(
        body,
        grid=(N // BR,),
        in_specs=[
            pl.BlockSpec((NCORES, BR, DIM), lambda i: (0, i, 0)),
            pl.BlockSpec((BR, DIM), lambda i: (i, 0)),
            pl.BlockSpec((BR, NCORES), lambda i: (i, 0)),
            pl.BlockSpec((1, DIM), lambda i: (0, 0)),
            pl.BlockSpec((NCORES, BH, DIM), lambda i: (0, i, 0)),
            pl.BlockSpec((BH, NCORES), lambda i: (i, 0)),
            pl.BlockSpec((DIM, DIM), lambda i: (0, 0)),
            pl.BlockSpec((1, DIM), lambda i: (0, 0)),
        ],
        out_specs=[pl.BlockSpec((BR, DIM), lambda i: (i, 0))] * 2
        + [pl.BlockSpec((BH, DIM), lambda i: (i, 0))],
        out_shape=[jax.ShapeDtypeStruct((N, DIM), _f32)] * 2
        + [jax.ShapeDtypeStruct((NHP, DIM), _f32)],
    )(S1, xw1, degT, b1_2, She, cheT, W_e, b_e2)


def _tc_final(S2, h, degT, Sm, cnodeT, h_in,
              W2, b2_2, W_v, b_v2, W_out, b_out2, W_lp, b_lp2):
    BR = 2000

    def body(S2_r, h_r, deg_r, Sm_r, cn_r, hin_r,
             W2_r, b2_r, Wv_r, bv_r, Wo_r, bo_r, Wlp_r, blp_r, out_r):
        deg = jnp.sum(deg_r[...], axis=1, keepdims=True) + 1.0
        dinv = lax.rsqrt(deg)
        agg2 = dinv * (S2_r[0] + S2_r[1]) + dinv * dinv * h_r[...]
        x_gnn = jnp.dot(agg2, W2_r[...], preferred_element_type=_f32) + b2_r[...]
        cnt = jnp.sum(cn_r[...], axis=1, keepdims=True)
        m = (Sm_r[0] + Sm_r[1]) * jnp.where(cnt > 0.0,
                                            1.0 / jnp.maximum(cnt, 1.0), 0.0)
        h2 = jnp.maximum(
            hin_r[...] + jnp.dot(m, Wv_r[...], preferred_element_type=_f32)
            + bv_r[...], 0.0)
        x_hyper = jnp.dot(h2, Wo_r[...], preferred_element_type=_f32) + bo_r[...]
        Wlp = Wlp_r[...]
        out_r[...] = (jnp.dot(x_gnn, Wlp[0:40], preferred_element_type=_f32)
                      + jnp.dot(x_hyper, Wlp[40:80], preferred_element_type=_f32)
                      + blp_r[...])

    return pl.pallas_call(
        body,
        grid=(N // BR,),
        in_specs=[
            pl.BlockSpec((NCORES, BR, DIM), lambda i: (0, i, 0)),
            pl.BlockSpec((BR, DIM), lambda i: (i, 0)),
            pl.BlockSpec((BR, NCORES), lambda i: (i, 0)),
            pl.BlockSpec((NCORES, BR, DIM), lambda i: (0, i, 0)),
            pl.BlockSpec((BR, NCORES), lambda i: (i, 0)),
            pl.BlockSpec((BR, DIM), lambda i: (i, 0)),
            pl.BlockSpec((DIM, 40), lambda i: (0, 0)),
            pl.BlockSpec((1, 40), lambda i: (0, 0)),
            pl.BlockSpec((DIM, DIM), lambda i: (0, 0)),
            pl.BlockSpec((1, DIM), lambda i: (0, 0)),
            pl.BlockSpec((DIM, 40), lambda i: (0, 0)),
            pl.BlockSpec((1, 40), lambda i: (0, 0)),
            pl.BlockSpec((80, 40), lambda i: (0, 0)),
            pl.BlockSpec((1, 40), lambda i: (0, 0)),
        ],
        out_specs=pl.BlockSpec((BR, 40), lambda i: (i, 0)),
        out_shape=jax.ShapeDtypeStruct((N, 40), _f32),
    )(S2, h, degT, Sm, cnodeT, h_in,
      W2, b2_2, W_v, b_v2, W_out, b_out2, W_lp, b_lp2)


def kernel(x, edge_index, hyperedge_index, W1, b1, W2, b2, W_in, b_in,
           W_e, b_e, W_v, b_v, W_out, b_out, W_lp, b_lp):
    npad = EP - E
    ar = jnp.arange(npad, dtype=jnp.int32)
    pad_g = ar % 256
    pad_n = N + (ar % (NP - N))
    pad_h = NH + (ar % (NHP - NH))
    src_g = jnp.concatenate([edge_index[0], pad_g])
    dst_n = jnp.concatenate([edge_index[1], pad_n])
    hnode_g = jnp.concatenate([hyperedge_index[0], pad_g])
    hnode_n = jnp.concatenate([hyperedge_index[0], pad_n])
    hnode_h = jnp.concatenate([hyperedge_index[0], pad_h])
    hhe_g = jnp.concatenate([hyperedge_index[1], pad_g])
    hhe_h = jnp.concatenate([hyperedge_index[1], pad_h])

    zeros640 = jnp.zeros((640,), _f32)
    onesC = jnp.ones((C,), _f32)
    zeros2d = jnp.zeros((C, DIM), _f32)

    deg_p, che_p, cnode_p = _sc_counts(dst_n, hhe_h, hnode_n, zeros640, onesC)
    degT = deg_p.reshape(NCORES, NP).T[:N]
    cheT = che_p.reshape(NCORES, NHP).T
    cnodeT = cnode_p.reshape(NCORES, NP).T[:N]

    y1, xw1, h_in = _tc_dense1(x, degT, W1, W_in, b_in.reshape(1, DIM))
    S1, She = _sc_scatter2(y1, src_g, dst_n, NP, h_in, hnode_g, hhe_h, NHP,
                           zeros2d)
    h, y2, e2 = _tc_dense23(S1[:, :N], xw1, degT, b1.reshape(1, DIM),
                            She, cheT, W_e, b_e.reshape(1, DIM))
    S2, Sm = _sc_scatter2(y2, src_g, dst_n, NP, e2, hhe_g, hnode_h, NHP,
                          zeros2d)
    Smp = jnp.concatenate([Sm, jnp.zeros((NCORES, N - NHP, DIM), _f32)],
                          axis=1)
    out = _tc_final(S2[:, :N], h, degT, Smp, cnodeT, h_in,
                    W2, b2.reshape(1, 40), W_v, b_v.reshape(1, DIM),
                    W_out, b_out.reshape(1, 40), W_lp, b_lp.reshape(1, 40))
    return out


# speedup vs baseline: 28.7933x; 1.0277x over previous
import functools

import jax
import jax.numpy as jnp
from jax import lax
from jax.experimental import pallas as pl
from jax.experimental.pallas import tpu as pltpu
from jax.experimental.pallas import tpu_sc as plsc

N = 10000
NP = 10240
NH = 5000
NHP = 5120
NMP = 6400
E = 320000
EP = 327680
DIM = 64
NCORES = 2
NSUB = 16
NW = NCORES * NSUB
EPW = EP // NW
C = 80
NCHUNK = EPW // C

_f32 = jnp.float32
D_FEAT_ = 128


def _mesh():
    return plsc.VectorSubcoreMesh(core_axis_name="c", subcore_axis_name="s")


def _sc_counts(dst, hhe, hnode, zeros640, onesC):
    out_type = [
        jax.ShapeDtypeStruct((NCORES * NP,), _f32),
        jax.ShapeDtypeStruct((NCORES * NHP,), _f32),
        jax.ShapeDtypeStruct((NCORES * NP,), _f32),
    ]

    @functools.partial(
        pl.kernel,
        out_type=out_type,
        mesh=_mesh(),
        scratch_types=[
            pltpu.VMEM((EPW,), jnp.int32),
            pltpu.VMEM((C,), _f32),
            pltpu.VMEM((640,), _f32),
            pltpu.VMEM_SHARED((NP,), _f32),
            pltpu.VMEM_SHARED((NHP,), _f32),
            pltpu.VMEM_SHARED((NP,), _f32),
            pltpu.SemaphoreType.DMA,
        ],
    )
    def k(dst_h, hhe_h, hnode_h, z_h, o_h, deg_o, che_o, cnode_o,
          idx_v, ones_v, stage_v, acc_deg, acc_he, acc_node, csem):
        c = lax.axis_index("c")
        s = lax.axis_index("s")
        wid = s * NCORES + c
        pltpu.sync_copy(z_h, stage_v)
        pltpu.sync_copy(o_h, ones_v)
        pltpu.sync_copy(stage_v, acc_deg.at[pl.ds(s * 640, 640)])
        pltpu.sync_copy(stage_v.at[pl.ds(0, 320)], acc_he.at[pl.ds(s * 320, 320)])
        pltpu.sync_copy(stage_v, acc_node.at[pl.ds(s * 640, 640)])
        plsc.subcore_barrier()
        base = wid * EPW
        NB = 4

        for ih, acc in ((dst_h, acc_deg), (hhe_h, acc_he), (hnode_h, acc_node)):
            pltpu.sync_copy(ih.at[pl.ds(pl.multiple_of(base, 8), EPW)], idx_v)

            def desc(kk, acc=acc):
                off = pl.multiple_of(kk * C, 8)
                return pltpu.make_async_copy(
                    ones_v, acc.at[idx_v.at[pl.ds(off, C)]], csem)

            for b in range(NB):
                desc(b).start(add=True)

            def body(g, _, desc=desc):
                desc(g).wait()
                desc(g + NB).start(add=True)
                return 0

            lax.fori_loop(0, NCHUNK - NB, body, 0)
            for b in range(NB):
                desc(NCHUNK - NB + b).wait()
        plsc.subcore_barrier()
        pltpu.sync_copy(acc_deg.at[pl.ds(s * 640, 640)], stage_v)
        pltpu.sync_copy(stage_v, deg_o.at[pl.ds(pl.multiple_of(c * NP + s * 640, 8), 640)])
        pltpu.sync_copy(acc_he.at[pl.ds(s * 320, 320)], stage_v.at[pl.ds(0, 320)])
        pltpu.sync_copy(stage_v.at[pl.ds(0, 320)],
                        che_o.at[pl.ds(pl.multiple_of(c * NHP + s * 320, 8), 320)])
        pltpu.sync_copy(acc_node.at[pl.ds(s * 640, 640)], stage_v)
        pltpu.sync_copy(stage_v, cnode_o.at[pl.ds(pl.multiple_of(c * NP + s * 640, 8), 640)])

    return k(dst, hhe, hnode, zeros640, onesC)


def _sc_scatter2(t1, g1, d1, nacc1, t2, g2, d2, nacc2, zeros2d):
    out_type = [
        jax.ShapeDtypeStruct((NCORES, nacc1, DIM), _f32),
        jax.ShapeDtypeStruct((NCORES, nacc2, DIM), _f32),
    ]
    rpt1 = nacc1 // NSUB
    rpt2 = nacc2 // NSUB

    @functools.partial(
        pl.kernel,
        out_type=out_type,
        mesh=_mesh(),
        compiler_params=pltpu.CompilerParams(use_tc_tiling_on_sc=False),
        scratch_types=[
            pltpu.VMEM((EPW,), jnp.int32),
            pltpu.VMEM((EPW,), jnp.int32),
            [pltpu.VMEM((C, DIM), _f32)] * 8,
            pltpu.VMEM((C, DIM), _f32),
            pltpu.VMEM_SHARED((nacc1, DIM), _f32),
            pltpu.VMEM_SHARED((nacc2, DIM), _f32),
            [pltpu.SemaphoreType.DMA] * 8,
            [pltpu.SemaphoreType.DMA] * 8,
        ],
    )
    def k(t1_h, g1_h, d1_h, t2_h, g2_h, d2_h, z_h, o1, o2,
          gidx, didx, rows, zb, acc1, acc2, gsem, ssem):
        c = lax.axis_index("c")
        s = lax.axis_index("s")
        wid = s * NCORES + c
        pltpu.sync_copy(z_h, zb)
        for j in range(rpt1 // C):
            pltpu.sync_copy(zb, acc1.at[pl.ds(s * rpt1 + j * C, C)])
        for j in range(rpt2 // C):
            pltpu.sync_copy(zb, acc2.at[pl.ds(s * rpt2 + j * C, C)])
        plsc.subcore_barrier()
        base = pl.multiple_of(wid * EPW, 8)
        NB = 8
        D = 4
        GRP = NCHUNK // NB
        for t_h, g_h, d_h, acc in ((t1_h, g1_h, d1_h, acc1),
                                   (t2_h, g2_h, d2_h, acc2)):
            pltpu.sync_copy(g_h.at[pl.ds(base, EPW)], gidx)
            pltpu.sync_copy(d_h.at[pl.ds(base, EPW)], didx)

            def gdesc(kk, b, t_h=t_h):
                off = pl.multiple_of(kk * C, 8)
                return pltpu.make_async_copy(
                    t_h.at[gidx.at[pl.ds(off, C)]], rows[b], gsem[b])

            def sdesc(kk, b, acc=acc):
                off = pl.multiple_of(kk * C, 8)
                return pltpu.make_async_copy(
                    rows[b], acc.at[didx.at[pl.ds(off, C)]], ssem[b])

            for b in range(NB):
                gdesc(b, b).start()
                if b >= D:
                    gdesc(b - D, b - D).wait()
                    sdesc(b - D, b - D).start(add=True)

            def body(g, _, gdesc=gdesc, sdesc=sdesc):
                for b in range(NB):
                    kk = g * NB + b
                    sdesc(kk - NB, b).wait()
                    gdesc(kk, b).start()
                    gdesc(kk - D, (b - D) % NB).wait()
                    sdesc(kk - D, (b - D) % NB).start(add=True)
                return 0

            lax.fori_loop(1, GRP, body, 0)
            last = (GRP - 1) * NB
            for b in range(D):
                sdesc(last + b, b).wait()
                gdesc(last + NB - D + b, (b - D) % NB).wait()
                sdesc(last + NB - D + b, (b - D) % NB).start(add=True)
            for b in range(D):
                sdesc(last + NB - D + b, (b - D) % NB).wait()
        plsc.subcore_barrier()
        for acc, o_h, rpt in ((acc1, o1, rpt1), (acc2, o2, rpt2)):
            for j in range(rpt // C):
                b0 = pl.multiple_of(s * rpt + j * C, 8)
                pltpu.sync_copy(acc.at[pl.ds(b0, C)], zb)
                pltpu.sync_copy(zb, o_h.at[c, pl.ds(b0, C)])

    return k(t1, g1, d1, t2, g2, d2, zeros2d)


def _tc_dense1(x, degT, W1, W_in, b_in2):
    BR = 2048

    def body(x_r, deg_r, W1_r, Win_r, bin_r, y1_r, xw1_r, hin_r):
        xb = x_r[...]
        xw1 = jnp.dot(xb, W1_r[...], preferred_element_type=_f32)
        deg = jnp.sum(jnp.transpose(deg_r[...]), axis=1, keepdims=True) + 1.0
        dinv = lax.rsqrt(deg)
        y1_r[...] = xw1 * dinv
        xw1_r[...] = xw1
        hin_r[...] = jnp.maximum(
            jnp.dot(xb, Win_r[...], preferred_element_type=_f32) + bin_r[...], 0.0)

    return pl.pallas_call(
        body,
        grid=(NP // BR,),
        in_specs=[
            pl.BlockSpec((BR, 128), lambda i: (i, 0)),
            pl.BlockSpec((NCORES, BR), lambda i: (0, i)),
            pl.BlockSpec((128, DIM), lambda i: (0, 0)),
            pl.BlockSpec((128, DIM), lambda i: (0, 0)),
            pl.BlockSpec((1, DIM), lambda i: (0, 0)),
        ],
        out_specs=[pl.BlockSpec((BR, DIM), lambda i: (i, 0))] * 3,
        out_shape=[jax.ShapeDtypeStruct((NP, DIM), _f32)] * 3,
    )(x, degT, W1, W_in, b_in2)


def _tc_dense23(S1, xw1, degT, b1_2, She, cheT, W_e, b_e2):
    BR = 2048
    BH = NHP // (NP // BR)

    def body(S1_r, xw1_r, deg_r, b1_r, S_r, c_r, We_r, be_r,
             h_r, y2_r, e2_r):
        Ss = S1_r[0] + S1_r[1]
        deg = jnp.sum(jnp.transpose(deg_r[...]), axis=1, keepdims=True) + 1.0
        dinv = lax.rsqrt(deg)
        h = jnp.maximum(dinv * Ss + dinv * dinv * xw1_r[...] + b1_r[...], 0.0)
        h_r[...] = h
        y2_r[...] = h * dinv
        Se = S_r[0] + S_r[1]
        cnt = jnp.sum(jnp.transpose(c_r[...]), axis=1, keepdims=True)
        e = Se / jnp.maximum(cnt, 1.0)
        e2_r[...] = jnp.maximum(
            jnp.dot(e, We_r[...], preferred_element_type=_f32) + be_r[...], 0.0)

    return pl.pallas_call(
        body,
        grid=(NP // BR,),
        in_specs=[
            pl.BlockSpec((NCORES, BR, DIM), lambda i: (0, i, 0)),
            pl.BlockSpec((BR, DIM), lambda i: (i, 0)),
            pl.BlockSpec((NCORES, BR), lambda i: (0, i)),
            pl.BlockSpec((1, DIM), lambda i: (0, 0)),
            pl.BlockSpec((NCORES, BH, DIM), lambda i: (0, i, 0)),
            pl.BlockSpec((NCORES, BH), lambda i: (0, i)),
            pl.BlockSpec((DIM, DIM), lambda i: (0, 0)),
            pl.BlockSpec((1, DIM), lambda i: (0, 0)),
        ],
        out_specs=[pl.BlockSpec((BR, DIM), lambda i: (i, 0))] * 2
        + [pl.BlockSpec((BH, DIM), lambda i: (i, 0))],
        out_shape=[jax.ShapeDtypeStruct((NP, DIM), _f32)] * 2
        + [jax.ShapeDtypeStruct((NHP, DIM), _f32)],
    )(S1, xw1, degT, b1_2, She, cheT, W_e, b_e2)


def _tc_final(S2, h, degT, Sm, cnodeT, h_in,
              W2, b2_2, W_v, b_v2, W_out, b_out2, W_lp, b_lp2):
    BR = 2048

    def body(S2_r, h_r, deg_r, Sm_r, cn_r, hin_r,
             W2_r, b2_r, Wv_r, bv_r, Wo_r, bo_r, Wlp_r, blp_r, out_r):
        deg = jnp.sum(jnp.transpose(deg_r[...]), axis=1, keepdims=True) + 1.0
        dinv = lax.rsqrt(deg)
        agg2 = dinv * (S2_r[0] + S2_r[1]) + dinv * dinv * h_r[...]
        x_gnn = jnp.dot(agg2, W2_r[...], preferred_element_type=_f32) + b2_r[...]
        cnt = jnp.sum(jnp.transpose(cn_r[...]), axis=1, keepdims=True)
        m = (Sm_r[0] + Sm_r[1]) * jnp.where(cnt > 0.0,
                                            1.0 / jnp.maximum(cnt, 1.0), 0.0)
        h2 = jnp.maximum(
            hin_r[...] + jnp.dot(m, Wv_r[...], preferred_element_type=_f32)
            + bv_r[...], 0.0)
        x_hyper = jnp.dot(h2, Wo_r[...], preferred_element_type=_f32) + bo_r[...]
        Wlp = Wlp_r[...]
        out_r[...] = (jnp.dot(x_gnn, Wlp[0:40], preferred_element_type=_f32)
                      + jnp.dot(x_hyper, Wlp[40:80], preferred_element_type=_f32)
                      + blp_r[...])

    return pl.pallas_call(
        body,
        grid=(NP // BR,),
        in_specs=[
            pl.BlockSpec((NCORES, BR, DIM), lambda i: (0, i, 0)),
            pl.BlockSpec((BR, DIM), lambda i: (i, 0)),
            pl.BlockSpec((NCORES, BR), lambda i: (0, i)),
            pl.BlockSpec((NCORES, BR, DIM), lambda i: (0, i, 0)),
            pl.BlockSpec((NCORES, BR), lambda i: (0, i)),
            pl.BlockSpec((BR, DIM), lambda i: (i, 0)),
            pl.BlockSpec((DIM, 40), lambda i: (0, 0)),
            pl.BlockSpec((1, 40), lambda i: (0, 0)),
            pl.BlockSpec((DIM, DIM), lambda i: (0, 0)),
            pl.BlockSpec((1, DIM), lambda i: (0, 0)),
            pl.BlockSpec((DIM, 40), lambda i: (0, 0)),
            pl.BlockSpec((1, 40), lambda i: (0, 0)),
            pl.BlockSpec((80, 40), lambda i: (0, 0)),
            pl.BlockSpec((1, 40), lambda i: (0, 0)),
        ],
        out_specs=pl.BlockSpec((BR, 40), lambda i: (i, 0)),
        out_shape=jax.ShapeDtypeStruct((NP, 40), _f32),
    )(S2, h, degT, Sm, cnodeT, h_in,
      W2, b2_2, W_v, b_v2, W_out, b_out2, W_lp, b_lp2)


def kernel(x, edge_index, hyperedge_index, W1, b1, W2, b2, W_in, b_in,
           W_e, b_e, W_v, b_v, W_out, b_out, W_lp, b_lp):
    npad = EP - E
    ar = jnp.arange(npad, dtype=jnp.int32)
    pad_g = ar % 256
    pad_n = N + (ar % (NP - N))
    pad_h = NH + (ar % (NHP - NH))
    src_g = jnp.concatenate([edge_index[0], pad_g])
    dst_n = jnp.concatenate([edge_index[1], pad_n])
    hnode_g = jnp.concatenate([hyperedge_index[0], pad_g])
    hnode_n = jnp.concatenate([hyperedge_index[0], pad_n])
    hnode_h = jnp.concatenate([hyperedge_index[0], pad_h])
    hhe_g = jnp.concatenate([hyperedge_index[1], pad_g])
    hhe_h = jnp.concatenate([hyperedge_index[1], pad_h])

    zeros640 = jnp.zeros((640,), _f32)
    onesC = jnp.ones((C,), _f32)
    zeros2d = jnp.zeros((C, DIM), _f32)

    deg_p, che_p, cnode_p = _sc_counts(dst_n, hhe_h, hnode_n, zeros640, onesC)
    deg2 = deg_p.reshape(NCORES, NP)
    che2 = che_p.reshape(NCORES, NHP)
    cnode2 = cnode_p.reshape(NCORES, NP)

    xp = jnp.concatenate([x, jnp.zeros((NP - N, D_FEAT_), _f32)])
    y1, xw1, h_in = _tc_dense1(xp, deg2, W1, W_in, b_in.reshape(1, DIM))
    S1, She = _sc_scatter2(y1, src_g, dst_n, NP, h_in, hnode_g, hhe_h, NHP,
                           zeros2d)
    h, y2, e2 = _tc_dense23(S1, xw1, deg2, b1.reshape(1, DIM),
                            She, che2, W_e, b_e.reshape(1, DIM))
    S2, Sm = _sc_scatter2(y2, src_g, dst_n, NP, e2, hhe_g, hnode_h, NHP,
                          zeros2d)
    Smp = jnp.concatenate([Sm, jnp.zeros((NCORES, NP - NHP, DIM), _f32)],
                          axis=1)
    out = _tc_final(S2, h, deg2, Smp, cnode2, h_in,
                    W2, b2.reshape(1, 40), W_v, b_v.reshape(1, DIM),
                    W_out, b_out.reshape(1, 40), W_lp, b_lp.reshape(1, 40))
    return out[:N]
